# baseline (speedup 1.0000x reference)
import jax
import jax.numpy as jnp
from jax.experimental import pallas as pl

NUM_CODES = 512
CODE_DIM = 32


def _split3(x):
    x1 = x.astype(jnp.bfloat16)
    r1 = x - x1.astype(jnp.float32)
    x2 = r1.astype(jnp.bfloat16)
    r2 = r1 - x2.astype(jnp.float32)
    x3 = r2.astype(jnp.bfloat16)
    return x1, x2, x3


def _vq_kernel(zt_ref, ct_ref, zqt_ref, idx_ref):
    _, D, S = zt_ref.shape
    N = NUM_CODES
    ct = ct_ref[:]
    cnorm2 = jnp.sum(ct * ct, axis=0, keepdims=True)
    h1, h2, h3 = _split3(-0.5 * cnorm2)
    c1, c2, c3 = _split3(ct)
    c_cat = jnp.concatenate([c1, c2, c1, c3, c1, c2, h1, h2, h3], axis=0)
    c12 = jnp.concatenate([c1, c2], axis=0)
    sub = jax.lax.broadcasted_iota(jnp.int32, (N, S), 0)
    x = zt_ref[0]
    x1, x2, x3 = _split3(x)
    one = jnp.ones((1, S), jnp.bfloat16)
    z_cat = jnp.concatenate([x1, x1, x2, x1, x3, x2, one, one, one],
                            axis=0)
    g = jax.lax.dot_general(
        c_cat, z_cat, (((0,), (0,)), ((), ())),
        preferred_element_type=jnp.float32)
    m = jnp.max(g, axis=0, keepdims=True)
    idxb = jnp.min(jnp.where(g == m, sub, N),
                   axis=0, keepdims=True)
    onehot = (sub == idxb).astype(jnp.bfloat16)
    zq2 = jax.lax.dot_general(
        c12, onehot, (((1,), (0,)), ((), ())),
        preferred_element_type=jnp.float32)
    zqt_ref[0] = zq2[:CODE_DIM] + zq2[CODE_DIM:]
    b = pl.program_id(0)
    idx_ref[pl.ds(b, 1), :] = idxb


def kernel(z_e, codebook):
    B, S, D = z_e.shape
    N = NUM_CODES
    zt = jnp.swapaxes(z_e, 1, 2)
    ct = codebook.T
    zqt, idx = pl.pallas_call(
        _vq_kernel,
        grid=(B,),
        in_specs=[
            pl.BlockSpec((1, D, S), lambda b: (b, 0, 0)),
            pl.BlockSpec((D, N), lambda b: (0, 0)),
        ],
        out_specs=[
            pl.BlockSpec((1, D, S), lambda b: (b, 0, 0)),
            pl.BlockSpec((B, S), lambda b: (0, 0)),
        ],
        out_shape=[
            jax.ShapeDtypeStruct((B, D, S), jnp.float32),
            jax.ShapeDtypeStruct((B, S), jnp.int32),
        ],
    )(zt, ct)
    return jnp.swapaxes(zqt, 1, 2), idx


# speedup vs baseline: 1.0719x; 1.0719x over previous
import jax
import jax.numpy as jnp
from jax.experimental import pallas as pl

NUM_CODES = 512
CODE_DIM = 32


def _split3(x):
    x1 = x.astype(jnp.bfloat16)
    r1 = x - x1.astype(jnp.float32)
    x2 = r1.astype(jnp.bfloat16)
    r2 = r1 - x2.astype(jnp.float32)
    x3 = r2.astype(jnp.bfloat16)
    return x1, x2, x3


def _vq_kernel(zt_ref, ct_ref, zqt_ref, idx_ref):
    _, D, S = zt_ref.shape
    N = NUM_CODES
    ct = ct_ref[:]
    cnorm2 = jnp.sum(ct * ct, axis=0, keepdims=True)
    h1, h2, h3 = _split3(-0.5 * cnorm2)
    c1, c2, c3 = _split3(ct)
    c_cat = jnp.concatenate([c1, c2, c1, c3, c1, c2, h1, h2, h3], axis=0)
    c12 = jnp.concatenate([c1, c2], axis=0)
    sub = jax.lax.broadcasted_iota(jnp.int32, (N, S), 0)
    gbase = pl.program_id(0) * 2
    for j in range(2):
        x = zt_ref[j]
        x1, x2, x3 = _split3(x)
        one = jnp.ones((1, S), jnp.bfloat16)
        z_cat = jnp.concatenate([x1, x1, x2, x1, x3, x2, one, one, one],
                                axis=0)
        g = jax.lax.dot_general(
            c_cat, z_cat, (((0,), (0,)), ((), ())),
            preferred_element_type=jnp.float32)
        m = jnp.max(g, axis=0, keepdims=True)
        idxb = jnp.min(jnp.where(g == m, sub, N),
                       axis=0, keepdims=True)
        onehot = (sub == idxb).astype(jnp.bfloat16)
        zq2 = jax.lax.dot_general(
            c12, onehot, (((1,), (0,)), ((), ())),
            preferred_element_type=jnp.float32)
        zqt_ref[j] = zq2[:CODE_DIM] + zq2[CODE_DIM:]
        idx_ref[pl.ds(gbase + j, 1), :] = idxb


def kernel(z_e, codebook):
    B, S, D = z_e.shape
    N = NUM_CODES
    zt = jnp.swapaxes(z_e, 1, 2)
    ct = codebook.T
    zqt, idx = pl.pallas_call(
        _vq_kernel,
        grid=(B // 2,),
        in_specs=[
            pl.BlockSpec((2, D, S), lambda b: (b, 0, 0)),
            pl.BlockSpec((D, N), lambda b: (0, 0)),
        ],
        out_specs=[
            pl.BlockSpec((2, D, S), lambda b: (b, 0, 0)),
            pl.BlockSpec((B, S), lambda b: (0, 0)),
        ],
        out_shape=[
            jax.ShapeDtypeStruct((B, D, S), jnp.float32),
            jax.ShapeDtypeStruct((B, S), jnp.int32),
        ],
    )(zt, ct)
    return jnp.swapaxes(zqt, 1, 2), idx


# speedup vs baseline: 1.1002x; 1.0263x over previous
import jax
import jax.numpy as jnp
from jax.experimental import pallas as pl

NUM_CODES = 512
CODE_DIM = 32


def _split3(x):
    x1 = x.astype(jnp.bfloat16)
    r1 = x - x1.astype(jnp.float32)
    x2 = r1.astype(jnp.bfloat16)
    r2 = r1 - x2.astype(jnp.float32)
    x3 = r2.astype(jnp.bfloat16)
    return x1, x2, x3


def _vq_kernel(zt_ref, ct_ref, zqt_ref, idx_ref):
    B, D, S = zt_ref.shape
    N = NUM_CODES
    ct = ct_ref[:]
    cnorm2 = jnp.sum(ct * ct, axis=0, keepdims=True)
    h1, h2, h3 = _split3(-0.5 * cnorm2)
    c1, c2, c3 = _split3(ct)
    c_cat = jnp.concatenate([c1, c2, c1, c3, c1, c2, h1, h2, h3], axis=0)
    c12 = jnp.concatenate([c1, c2], axis=0)
    sub = jax.lax.broadcasted_iota(jnp.int32, (N, S), 0)
    for b in range(B):
        x = zt_ref[b]
        x1, x2, x3 = _split3(x)
        one = jnp.ones((1, S), jnp.bfloat16)
        z_cat = jnp.concatenate([x1, x1, x2, x1, x3, x2, one, one, one],
                                axis=0)
        g = jax.lax.dot_general(
            c_cat, z_cat, (((0,), (0,)), ((), ())),
            preferred_element_type=jnp.float32)
        m = jnp.max(g, axis=0, keepdims=True)
        idxb = jnp.min(jnp.where(g == m, sub, N),
                       axis=0, keepdims=True)
        onehot = (sub == idxb).astype(jnp.bfloat16)
        zq2 = jax.lax.dot_general(
            c12, onehot, (((1,), (0,)), ((), ())),
            preferred_element_type=jnp.float32)
        zqt_ref[b] = zq2[:D] + zq2[D:]
        idx_ref[pl.ds(b, 1), :] = idxb


def kernel(z_e, codebook):
    B, S, D = z_e.shape
    zt = jnp.swapaxes(z_e, 1, 2)
    ct = codebook.T
    zqt, idx = pl.pallas_call(
        _vq_kernel,
        out_shape=[
            jax.ShapeDtypeStruct((B, D, S), jnp.float32),
            jax.ShapeDtypeStruct((B, S), jnp.int32),
        ],
    )(zt, ct)
    return jnp.swapaxes(zqt, 1, 2), idx
